# baseline (speedup 1.0000x reference)
import functools

import jax
import jax.numpy as jnp
import numpy as np
from jax.experimental import pallas as pl


_TWO_PI = 2.0 * np.pi
_DIR_OFFSET = 0.78539


def _body(bp_ref, rt_ref, an_ref, out_ref):
    bpT = jnp.transpose(bp_ref[...])
    rtT = jnp.transpose(rt_ref[...])
    anT = jnp.transpose(an_ref[...])

    bp6 = bpT[6:7, :]
    rt6 = rtT[6:7, :]
    an6 = anT[6:7, :]

    suv = jnp.sin(jnp.concatenate([bp6 + rt6, bp6 - rt6], axis=0))
    u = suv[0:1, :]
    v = suv[1:2, :]
    s1 = (u + v) * 0.5
    s2 = (u - v) * 0.5

    x = rt6 + an6 - _DIR_OFFSET
    m = x - jnp.floor(x / _TWO_PI) * _TWO_PI
    d = jnp.clip(jnp.floor(m / np.pi), 0.0, 1.0)

    outT = jnp.concatenate(
        [bpT[:6], s1, rtT[:6], s2, 1.0 - d, d], axis=0)
    out_ref[...] = jnp.transpose(outT)


@functools.partial(jax.jit, static_argnames=("block_rows",))
def _run(bp, rt, an, block_rows):
    B, N, _ = bp.shape
    grid = (B, N // block_rows)
    in_spec = pl.BlockSpec((None, block_rows, 7), lambda b, i: (b, i, 0))
    out_spec = pl.BlockSpec((None, block_rows, 16), lambda b, i: (b, i, 0))
    return pl.pallas_call(
        _body,
        grid=grid,
        in_specs=[in_spec, in_spec, in_spec],
        out_specs=out_spec,
        out_shape=jax.ShapeDtypeStruct((B, N, 16), bp.dtype),
    )(bp, rt, an)


def kernel(box_preds, reg_targets, anchors):
    return _run(box_preds, reg_targets, anchors, 4224)


# speedup vs baseline: 2.6671x; 2.6671x over previous
import functools

import jax
import jax.numpy as jnp
import numpy as np
from jax.experimental import pallas as pl


_TWO_PI = 2.0 * np.pi
_DIR_OFFSET = 0.78539


def _body(a_ref, b_ref, c_ref, s1_ref, s2_ref, d14_ref, d15_ref):
    a = a_ref[...]
    b = b_ref[...]
    c = c_ref[...]

    suv = jnp.sin(jnp.concatenate([a + b, a - b], axis=0))
    u = suv[: a.shape[0]]
    v = suv[a.shape[0]:]
    s1_ref[...] = (u + v) * 0.5
    s2_ref[...] = (u - v) * 0.5

    x = b + c - _DIR_OFFSET
    m = x - jnp.floor(x / _TWO_PI) * _TWO_PI
    d = jnp.clip(jnp.floor(m / np.pi), 0.0, 1.0)
    d15_ref[...] = d
    d14_ref[...] = 1.0 - d


@functools.partial(jax.jit, static_argnames=("block_lanes",))
def _run(bp6, rt6, an6, block_lanes):
    B, N = bp6.shape
    grid = (N // block_lanes,)
    spec = pl.BlockSpec((B, block_lanes), lambda i: (0, i))
    plane = jax.ShapeDtypeStruct((B, N), bp6.dtype)
    return pl.pallas_call(
        _body,
        grid=grid,
        in_specs=[spec, spec, spec],
        out_specs=[spec, spec, spec, spec],
        out_shape=[plane, plane, plane, plane],
    )(bp6, rt6, an6)


def kernel(box_preds, reg_targets, anchors):
    bp6 = box_preds[:, :, 6]
    rt6 = reg_targets[:, :, 6]
    an6 = anchors[:, :, 6]
    s1, s2, d14, d15 = _run(bp6, rt6, an6, 8448)
    return jnp.concatenate(
        [box_preds[:, :, :6], s1[:, :, None],
         reg_targets[:, :, :6], s2[:, :, None],
         d14[:, :, None], d15[:, :, None]], axis=2)


# speedup vs baseline: 25.0643x; 9.3975x over previous
import functools

import jax
import jax.numpy as jnp
import numpy as np
from jax.experimental import pallas as pl


_TWO_PI = 2.0 * np.pi
_DIR_OFFSET = 0.78539


def _body(bp_ref, rt_ref, an_ref, out_ref):
    a = bp_ref[6]
    b = rt_ref[6]
    c = an_ref[6]

    suv = jnp.sin(jnp.concatenate([a + b, a - b], axis=0))
    u = suv[:4]
    v = suv[4:]
    s1 = (u + v) * 0.5
    s2 = (u - v) * 0.5

    x = b + c - _DIR_OFFSET
    m = x - jnp.floor(x / _TWO_PI) * _TWO_PI
    d = jnp.clip(jnp.floor(m / np.pi), 0.0, 1.0)

    for i in range(4):
        out_ref[i] = jnp.concatenate(
            [bp_ref[0:6, i], s1[i:i + 1], rt_ref[0:6, i], s2[i:i + 1],
             1.0 - d[i:i + 1], d[i:i + 1]], axis=0)


@functools.partial(jax.jit, static_argnames=("block_lanes",))
def _run(bpP, rtP, anP, block_lanes):
    C, B, N = bpP.shape
    grid = (N // block_lanes,)
    in_spec = pl.BlockSpec((C, B, block_lanes), lambda i: (0, 0, i))
    out_spec = pl.BlockSpec((B, 16, block_lanes), lambda i: (0, 0, i))
    return pl.pallas_call(
        _body,
        grid=grid,
        in_specs=[in_spec, in_spec, in_spec],
        out_specs=out_spec,
        out_shape=jax.ShapeDtypeStruct((B, 16, N), bpP.dtype),
    )(bpP, rtP, anP)


def kernel(box_preds, reg_targets, anchors):
    bpP = jnp.transpose(box_preds, (2, 0, 1))
    rtP = jnp.transpose(reg_targets, (2, 0, 1))
    anP = jnp.transpose(anchors, (2, 0, 1))
    outP = _run(bpP, rtP, anP, 8448)
    return jnp.transpose(outP, (0, 2, 1))
